# baseline (speedup 1.0000x reference)
import functools

import jax
import jax.numpy as jnp
from jax.experimental import pallas as pl

FB = 5


def _encode_kernel(x_ref, W1c_ref, W2c_ref, b1T_ref, b2T_ref, eid_ref,
                   out_ref, E):
    eidr = eid_ref[...]
    masks = [(eidr == e).astype(jnp.bfloat16) for e in range(E)]
    onehot = jnp.concatenate(masks, axis=0).astype(jnp.float32)
    b1L = jnp.dot(b1T_ref[...], onehot,
                  preferred_element_type=jnp.float32)
    b2L = jnp.dot(b2T_ref[...], onehot,
                  preferred_element_type=jnp.float32)
    for f in range(FB):
        xf = x_ref[f].astype(jnp.bfloat16)
        xstack = jnp.concatenate(
            [xf * m for m in masks], axis=0)
        h = jnp.dot(W1c_ref[...], xstack,
                    preferred_element_type=jnp.float32) + b1L
        a = (h / (1.0 + jnp.abs(h))).astype(jnp.bfloat16)
        astack = jnp.concatenate(
            [a * m for m in masks], axis=0)
        o = jnp.dot(W2c_ref[...], astack,
                    preferred_element_type=jnp.float32)
        out_ref[f] = o + b2L


@jax.jit
def kernel(x, Ws1, bs1, Ws2, bs2, eid):
    B, MAX_F, N = x.shape
    E, _, H = Ws1.shape
    P = Ws2.shape[-1]

    xt = jnp.transpose(x, (1, 2, 0))
    W1c = (jnp.transpose(Ws1, (2, 0, 1)).reshape(H, E * N)
           .astype(jnp.bfloat16))
    W2c = (jnp.transpose(Ws2, (2, 0, 1)).reshape(P, E * H)
           .astype(jnp.bfloat16))
    eid2 = eid.reshape(1, B)

    grid = MAX_F // FB
    outT = pl.pallas_call(
        functools.partial(_encode_kernel, E=E),
        grid=(grid,),
        in_specs=[
            pl.BlockSpec((FB, N, B), lambda i: (i, 0, 0)),
            pl.BlockSpec((H, E * N), lambda i: (0, 0)),
            pl.BlockSpec((P, E * H), lambda i: (0, 0)),
            pl.BlockSpec((H, E), lambda i: (0, 0)),
            pl.BlockSpec((P, E), lambda i: (0, 0)),
            pl.BlockSpec((1, B), lambda i: (0, 0)),
        ],
        out_specs=pl.BlockSpec((FB, P, B), lambda i: (i, 0, 0)),
        out_shape=jax.ShapeDtypeStruct((MAX_F, P, B), jnp.float32),
    )(xt, W1c, W2c, bs1.T, bs2.T, eid2)

    return jnp.transpose(outT, (2, 0, 1))


# speedup vs baseline: 1.0226x; 1.0226x over previous
import functools

import jax
import jax.numpy as jnp
from jax.experimental import pallas as pl

FB = 10


def _encode_kernel(x_ref, W1c_ref, W2c_ref, b1T_ref, b2T_ref, eid_ref,
                   out_ref, E):
    eidr = eid_ref[...]
    masks = [(eidr == e).astype(jnp.bfloat16) for e in range(E)]
    onehot = jnp.concatenate(masks, axis=0).astype(jnp.float32)
    b1L = jnp.dot(b1T_ref[...], onehot,
                  preferred_element_type=jnp.float32)
    b2L = jnp.dot(b2T_ref[...], onehot,
                  preferred_element_type=jnp.float32)
    for f in range(FB):
        xf = x_ref[f].astype(jnp.bfloat16)
        xstack = jnp.concatenate(
            [xf * m for m in masks], axis=0)
        h = jnp.dot(W1c_ref[...], xstack,
                    preferred_element_type=jnp.float32) + b1L
        a = (h / (1.0 + jnp.abs(h))).astype(jnp.bfloat16)
        astack = jnp.concatenate(
            [a * m for m in masks], axis=0)
        o = jnp.dot(W2c_ref[...], astack,
                    preferred_element_type=jnp.float32)
        out_ref[f] = o + b2L


@jax.jit
def kernel(x, Ws1, bs1, Ws2, bs2, eid):
    B, MAX_F, N = x.shape
    E, _, H = Ws1.shape
    P = Ws2.shape[-1]

    xt = jnp.transpose(x, (1, 2, 0))
    W1c = (jnp.transpose(Ws1, (2, 0, 1)).reshape(H, E * N)
           .astype(jnp.bfloat16))
    W2c = (jnp.transpose(Ws2, (2, 0, 1)).reshape(P, E * H)
           .astype(jnp.bfloat16))
    eid2 = eid.reshape(1, B)

    grid = MAX_F // FB
    outT = pl.pallas_call(
        functools.partial(_encode_kernel, E=E),
        grid=(grid,),
        in_specs=[
            pl.BlockSpec((FB, N, B), lambda i: (i, 0, 0)),
            pl.BlockSpec((H, E * N), lambda i: (0, 0)),
            pl.BlockSpec((P, E * H), lambda i: (0, 0)),
            pl.BlockSpec((H, E), lambda i: (0, 0)),
            pl.BlockSpec((P, E), lambda i: (0, 0)),
            pl.BlockSpec((1, B), lambda i: (0, 0)),
        ],
        out_specs=pl.BlockSpec((FB, P, B), lambda i: (i, 0, 0)),
        out_shape=jax.ShapeDtypeStruct((MAX_F, P, B), jnp.float32),
    )(xt, W1c, W2c, bs1.T, bs2.T, eid2)

    return jnp.transpose(outT, (2, 0, 1))
